# baseline (speedup 1.0000x reference)
import jax
import jax.numpy as jnp
from jax import lax
from jax.experimental import pallas as pl
from jax.experimental.pallas import tpu as pltpu
from jax.experimental.pallas import tpu_sc as plsc

_LANES = 16
_NC = 2
_NS = 16
_CHUNK = 128


def _make_sc_kernel(B, NNZ, FD, SD, interpret=False):
  n_workers = _NC * _NS
  rpw = B // n_workers
  assert B % n_workers == 0 and rpw % _CHUNK == 0 and NNZ % _LANES == 0

  mesh = plsc.VectorSubcoreMesh(
      core_axis_name="c", subcore_axis_name="s",
      num_cores=_NC, num_subcores=_NS)

  n_chunks = rpw // _CHUNK

  def sc_body(ff, gi, gv, ai, av, ftab, gtab, atab,
              f_out, g_out, a_out, mg_out, ma_out,
              sidx, sval, valbuf, srcbuf, mbuf, rows, sem, *gidx):
    wid = lax.axis_index("s") * _NC + lax.axis_index("c")
    base = wid * rpw

    def gather_rows(tab):
      descs = [
          pltpu.async_copy(tab.at[gidx[k]],
                           rows.at[pl.ds(k * _CHUNK, _CHUNK)], sem)
          for k in range(n_chunks)
      ]
      for d in descs:
        d.wait()

    for k in range(n_chunks):
      pltpu.sync_copy(ff.at[pl.ds(base + k * _CHUNK, _CHUNK)], gidx[k])
    gather_rows(ftab)
    pltpu.sync_copy(rows, f_out.at[pl.ds(base, rpw)])

    def zbody(i, _):
      srcbuf[pl.ds(i * _LANES, _LANES)] = jnp.zeros((_LANES,), jnp.int32)
      return 0
    lax.fori_loop(0, rpw // _LANES, zbody, 0)

    for ih, vh, code in ((gi, gv, 1), (ai, av, 2)):
      pltpu.sync_copy(ih, sidx)
      pltpu.sync_copy(vh, sval)
      code16 = jnp.full((_LANES,), code, jnp.int32)

      def p1body(i, _, code16=code16):
        idx16 = sidx[pl.ds(i * _LANES, _LANES)]
        val16 = sval[pl.ds(i * _LANES, _LANES)]
        inb = (idx16 >= base) & (idx16 < base + rpw)
        local = jnp.where(inb, idx16 - base, 0)
        _, win = plsc.scan_count(local, mask=inb)
        plsc.store_scatter(valbuf, [local], val16, mask=win)
        plsc.store_scatter(srcbuf, [local], code16, mask=win)
        return 0
      lax.fori_loop(0, NNZ // _LANES, p1body, 0)

    for tab, out_hbm, m_out, code in ((gtab, g_out, mg_out, 1),
                                      (atab, a_out, ma_out, 2)):
      code16 = jnp.full((_LANES,), code, jnp.int32)

      for i in range(rpw // _LANES):
        v = valbuf[pl.ds(i * _LANES, _LANES)]
        s = srcbuf[pl.ds(i * _LANES, _LANES)]
        sel = s == code16
        lo = (i * _LANES) % _CHUNK
        gidx[(i * _LANES) // _CHUNK][pl.ds(lo, _LANES)] = jnp.where(sel, v, 0)
        mbuf[pl.ds(i * _LANES, _LANES)] = sel.astype(jnp.float32)
      gather_rows(tab)
      pltpu.sync_copy(rows, out_hbm.at[pl.ds(base, rpw)])
      pltpu.sync_copy(mbuf, m_out.at[pl.ds(base, rpw)])

  return pl.kernel(
      sc_body,
      out_type=[
          jax.ShapeDtypeStruct((B, FD), jnp.float32),
          jax.ShapeDtypeStruct((B, SD), jnp.float32),
          jax.ShapeDtypeStruct((B, SD), jnp.float32),
          jax.ShapeDtypeStruct((B,), jnp.float32),
          jax.ShapeDtypeStruct((B,), jnp.float32),
      ],
      mesh=mesh,
      scratch_types=[
          pltpu.VMEM((NNZ,), jnp.int32),
          pltpu.VMEM((NNZ,), jnp.int32),
          pltpu.VMEM((rpw,), jnp.int32),
          pltpu.VMEM((rpw,), jnp.int32),
          pltpu.VMEM((rpw,), jnp.float32),
          pltpu.VMEM((rpw, FD), jnp.float32),
          pltpu.SemaphoreType.DMA,
      ] + [pltpu.VMEM((_CHUNK,), jnp.int32) for _ in range(rpw // _CHUNK)],
      compiler_params=pltpu.CompilerParams(
          needs_layout_passes=False, use_tc_tiling_on_sc=False),
      interpret=interpret)


def _make_tc_kernel(B, FD, SD, OD, blk, interpret=False):
  def tc_body(f, g, a, mg, ma, w, b, o):
    wv = w[:]
    s = g[:] * mg[:] + a[:] * ma[:]
    acc = jnp.dot(f[:], wv[:FD, :], preferred_element_type=jnp.float32)
    acc = acc + jnp.dot(s, wv[FD:, :], preferred_element_type=jnp.float32)
    o[:] = acc + b[:]

  return pl.pallas_call(
      tc_body,
      grid=(B // blk,),
      in_specs=[
          pl.BlockSpec((blk, FD), lambda i: (i, 0)),
          pl.BlockSpec((blk, SD), lambda i: (i, 0)),
          pl.BlockSpec((blk, SD), lambda i: (i, 0)),
          pl.BlockSpec((blk, 1), lambda i: (i, 0)),
          pl.BlockSpec((blk, 1), lambda i: (i, 0)),
          pl.BlockSpec((FD + SD, OD), lambda i: (0, 0)),
          pl.BlockSpec((1, OD), lambda i: (0, 0)),
      ],
      out_specs=pl.BlockSpec((blk, OD), lambda i: (i, 0)),
      out_shape=jax.ShapeDtypeStruct((B, OD), jnp.float32),
      interpret=interpret)


def _run(fixed_features, geom_index, geom_value, attr_index, attr_value,
         fixed_table, geom_table, attr_table, W, b, interpret=False):
  B = fixed_features.shape[0]
  NNZ = geom_index.shape[0]
  FD = fixed_table.shape[1]
  SD = geom_table.shape[1]
  OD = W.shape[1]
  ff = fixed_features.astype(jnp.int32)
  gi = geom_index.astype(jnp.int32)
  gv = geom_value.astype(jnp.int32)
  ai = attr_index.astype(jnp.int32)
  av = attr_value.astype(jnp.int32)
  f_emb, g_rows, a_rows, mg, ma = _make_sc_kernel(B, NNZ, FD, SD, interpret)(
      ff, gi, gv, ai, av, fixed_table, geom_table, attr_table)
  return _make_tc_kernel(B, FD, SD, OD, min(2048, B), interpret)(
      f_emb, g_rows, a_rows, mg.reshape(B, 1), ma.reshape(B, 1),
      W, b.reshape(1, OD))


def kernel(fixed_features, geom_index, geom_value, attr_index, attr_value,
           fixed_table, geom_table, attr_table, W, b):
  return _run(fixed_features, geom_index, geom_value, attr_index, attr_value,
              fixed_table, geom_table, attr_table, W, b)


# speedup vs baseline: 1.0008x; 1.0008x over previous
import jax
import jax.numpy as jnp
from jax import lax
from jax.experimental import pallas as pl
from jax.experimental.pallas import tpu as pltpu
from jax.experimental.pallas import tpu_sc as plsc

_LANES = 16
_NC = 2
_NS = 16
_CHUNK = 128


def _make_sc_kernel(B, NNZ, FD, SD, interpret=False):
  n_workers = _NC * _NS
  rpw = B // n_workers
  assert B % n_workers == 0 and rpw % _CHUNK == 0 and NNZ % _LANES == 0

  mesh = plsc.VectorSubcoreMesh(
      core_axis_name="c", subcore_axis_name="s",
      num_cores=_NC, num_subcores=_NS)

  n_chunks = rpw // _CHUNK

  def sc_body(ff, gi, gv, ai, av, ftab, gtab, atab,
              f_out, g_out, a_out, mg_out, ma_out,
              sidx, sval, valbuf, srcbuf, mbuf, rows, sem, *gidx):
    wid = lax.axis_index("s") * _NC + lax.axis_index("c")
    base = wid * rpw

    def gather_rows(tab):
      descs = [
          pltpu.async_copy(tab.at[gidx[k]],
                           rows.at[pl.ds(k * _CHUNK, _CHUNK)], sem)
          for k in range(n_chunks)
      ]
      for d in descs:
        d.wait()

    with jax.named_scope("fixed_gather"):
      for k in range(n_chunks):
        pltpu.sync_copy(ff.at[pl.ds(base + k * _CHUNK, _CHUNK)], gidx[k])
      gather_rows(ftab)
      pltpu.sync_copy(rows, f_out.at[pl.ds(base, rpw)])

    with jax.named_scope("initbuf"):
      def zbody(i, _):
        srcbuf[pl.ds(i * _LANES, _LANES)] = jnp.zeros((_LANES,), jnp.int32)
        return 0
      lax.fori_loop(0, rpw // _LANES, zbody, 0)

    with jax.named_scope("winner_resolve"):
      for ih, vh, code in ((gi, gv, 1), (ai, av, 2)):
        pltpu.sync_copy(ih, sidx)
        pltpu.sync_copy(vh, sval)
        code16 = jnp.full((_LANES,), code, jnp.int32)

        def p1body(i, _, code16=code16):
          idx16 = sidx[pl.ds(i * _LANES, _LANES)]
          val16 = sval[pl.ds(i * _LANES, _LANES)]
          inb = (idx16 >= base) & (idx16 < base + rpw)
          local = jnp.where(inb, idx16 - base, 0)
          _, win = plsc.scan_count(local, mask=inb)
          plsc.store_scatter(valbuf, [local], val16, mask=win)
          plsc.store_scatter(srcbuf, [local], code16, mask=win)
          return 0
        lax.fori_loop(0, NNZ // _LANES, p1body, 0)

    for tab, out_hbm, m_out, code in ((gtab, g_out, mg_out, 1),
                                      (atab, a_out, ma_out, 2)):
      code16 = jnp.full((_LANES,), code, jnp.int32)

      with jax.named_scope(f"sparse_gather_{code}"):
        for i in range(rpw // _LANES):
          v = valbuf[pl.ds(i * _LANES, _LANES)]
          s = srcbuf[pl.ds(i * _LANES, _LANES)]
          sel = s == code16
          lo = (i * _LANES) % _CHUNK
          gidx[(i * _LANES) // _CHUNK][pl.ds(lo, _LANES)] = jnp.where(sel, v, 0)
          mbuf[pl.ds(i * _LANES, _LANES)] = sel.astype(jnp.float32)
        gather_rows(tab)
        pltpu.sync_copy(rows, out_hbm.at[pl.ds(base, rpw)])
        pltpu.sync_copy(mbuf, m_out.at[pl.ds(base, rpw)])

  return pl.kernel(
      sc_body,
      out_type=[
          jax.ShapeDtypeStruct((B, FD), jnp.float32),
          jax.ShapeDtypeStruct((B, SD), jnp.float32),
          jax.ShapeDtypeStruct((B, SD), jnp.float32),
          jax.ShapeDtypeStruct((B,), jnp.float32),
          jax.ShapeDtypeStruct((B,), jnp.float32),
      ],
      mesh=mesh,
      scratch_types=[
          pltpu.VMEM((NNZ,), jnp.int32),
          pltpu.VMEM((NNZ,), jnp.int32),
          pltpu.VMEM((rpw,), jnp.int32),
          pltpu.VMEM((rpw,), jnp.int32),
          pltpu.VMEM((rpw,), jnp.float32),
          pltpu.VMEM((rpw, FD), jnp.float32),
          pltpu.SemaphoreType.DMA,
      ] + [pltpu.VMEM((_CHUNK,), jnp.int32) for _ in range(rpw // _CHUNK)],
      compiler_params=pltpu.CompilerParams(
          needs_layout_passes=False, use_tc_tiling_on_sc=False),
      interpret=interpret)


def _make_tc_kernel(B, FD, SD, OD, blk, interpret=False):
  def tc_body(f, g, a, mg, ma, w, b, o):
    wv = w[:]
    s = g[:] * mg[:] + a[:] * ma[:]
    acc = jnp.dot(f[:], wv[:FD, :], preferred_element_type=jnp.float32)
    acc = acc + jnp.dot(s, wv[FD:, :], preferred_element_type=jnp.float32)
    o[:] = acc + b[:]

  return pl.pallas_call(
      tc_body,
      grid=(B // blk,),
      in_specs=[
          pl.BlockSpec((blk, FD), lambda i: (i, 0)),
          pl.BlockSpec((blk, SD), lambda i: (i, 0)),
          pl.BlockSpec((blk, SD), lambda i: (i, 0)),
          pl.BlockSpec((blk, 1), lambda i: (i, 0)),
          pl.BlockSpec((blk, 1), lambda i: (i, 0)),
          pl.BlockSpec((FD + SD, OD), lambda i: (0, 0)),
          pl.BlockSpec((1, OD), lambda i: (0, 0)),
      ],
      out_specs=pl.BlockSpec((blk, OD), lambda i: (i, 0)),
      out_shape=jax.ShapeDtypeStruct((B, OD), jnp.float32),
      interpret=interpret)


def _run(fixed_features, geom_index, geom_value, attr_index, attr_value,
         fixed_table, geom_table, attr_table, W, b, interpret=False):
  B = fixed_features.shape[0]
  NNZ = geom_index.shape[0]
  FD = fixed_table.shape[1]
  SD = geom_table.shape[1]
  OD = W.shape[1]
  ff = fixed_features.astype(jnp.int32)
  gi = geom_index.astype(jnp.int32)
  gv = geom_value.astype(jnp.int32)
  ai = attr_index.astype(jnp.int32)
  av = attr_value.astype(jnp.int32)
  f_emb, g_rows, a_rows, mg, ma = _make_sc_kernel(B, NNZ, FD, SD, interpret)(
      ff, gi, gv, ai, av, fixed_table, geom_table, attr_table)
  return _make_tc_kernel(B, FD, SD, OD, min(2048, B), interpret)(
      f_emb, g_rows, a_rows, mg.reshape(B, 1), ma.reshape(B, 1),
      W, b.reshape(1, OD))


def kernel(fixed_features, geom_index, geom_value, attr_index, attr_value,
           fixed_table, geom_table, attr_table, W, b):
  return _run(fixed_features, geom_index, geom_value, attr_index, attr_value,
              fixed_table, geom_table, attr_table, W, b)


# speedup vs baseline: 1.5080x; 1.5068x over previous
import jax
import jax.numpy as jnp
from jax import lax
from jax.experimental import pallas as pl
from jax.experimental.pallas import tpu as pltpu
from jax.experimental.pallas import tpu_sc as plsc

_LANES = 16
_NC = 2
_NS = 16
_CHUNK = 128


def _make_sc_kernel(B, NNZ, FD, SD, interpret=False):
  n_workers = _NC * _NS
  rpw = B // n_workers
  assert B % n_workers == 0 and rpw % _CHUNK == 0 and NNZ % _LANES == 0

  mesh = plsc.VectorSubcoreMesh(
      core_axis_name="c", subcore_axis_name="s",
      num_cores=_NC, num_subcores=_NS)

  n_chunks = rpw // _CHUNK

  def sc_body(ff, gi, gv, ai, av, ftab, gtab, atab,
              f_out, g_out, a_out, mg_out, ma_out,
              sidx, sval, valbuf, srcbuf, mbuf, rows, sem, *gidx):
    wid = lax.axis_index("s") * _NC + lax.axis_index("c")
    base = wid * rpw

    def gather_rows(tab):
      descs = [
          pltpu.async_copy(tab.at[gidx[k]],
                           rows.at[pl.ds(k * _CHUNK, _CHUNK)], sem)
          for k in range(n_chunks)
      ]
      for d in descs:
        d.wait()

    with jax.named_scope("fixed_gather"):
      for k in range(n_chunks):
        pltpu.sync_copy(ff.at[pl.ds(base + k * _CHUNK, _CHUNK)], gidx[k])
      gather_rows(ftab)
      pltpu.sync_copy(rows, f_out.at[pl.ds(base, rpw)])

    with jax.named_scope("initbuf"):
      def zbody(i, _):
        srcbuf[pl.ds(i * _LANES, _LANES)] = jnp.zeros((_LANES,), jnp.int32)
        return 0
      lax.fori_loop(0, rpw // _LANES, zbody, 0)

    with jax.named_scope("winner_resolve"):
      for ih, vh, code in ((gi, gv, 1), (ai, av, 2)):
        pltpu.sync_copy(ih, sidx)
        pltpu.sync_copy(vh, sval)
        code16 = jnp.full((_LANES,), code, jnp.int32)

        def p1body(i, _, code16=code16):
          idx16 = sidx[pl.ds(i * _LANES, _LANES)]
          val16 = sval[pl.ds(i * _LANES, _LANES)]
          inb = (idx16 >= base) & (idx16 < base + rpw)
          local = jnp.where(inb, idx16 - base, 0)
          _, win = plsc.scan_count(local, mask=inb)
          plsc.store_scatter(valbuf, [local], val16, mask=win)
          plsc.store_scatter(srcbuf, [local], code16, mask=win)
          return 0
        lax.fori_loop(0, NNZ // _LANES, p1body, 0)

    for tab, out_hbm, m_out, code in ((gtab, g_out, mg_out, 1),
                                      (atab, a_out, ma_out, 2)):
      code16 = jnp.full((_LANES,), code, jnp.int32)

      with jax.named_scope(f"sparse_gather_{code}"):
        lane16 = lax.iota(jnp.int32, _LANES)
        for i in range(rpw // _LANES):
          v = valbuf[pl.ds(i * _LANES, _LANES)]
          s = srcbuf[pl.ds(i * _LANES, _LANES)]
          sel = s == code16
          lo = (i * _LANES) % _CHUNK
          gidx[(i * _LANES) // _CHUNK][pl.ds(lo, _LANES)] = jnp.where(
              sel, v, base + lane16 + i * _LANES)
          mbuf[pl.ds(i * _LANES, _LANES)] = sel.astype(jnp.float32)
        gather_rows(tab)
        pltpu.sync_copy(rows, out_hbm.at[pl.ds(base, rpw)])
        pltpu.sync_copy(mbuf, m_out.at[pl.ds(base, rpw)])

  return pl.kernel(
      sc_body,
      out_type=[
          jax.ShapeDtypeStruct((B, FD), jnp.float32),
          jax.ShapeDtypeStruct((B, SD), jnp.float32),
          jax.ShapeDtypeStruct((B, SD), jnp.float32),
          jax.ShapeDtypeStruct((B,), jnp.float32),
          jax.ShapeDtypeStruct((B,), jnp.float32),
      ],
      mesh=mesh,
      scratch_types=[
          pltpu.VMEM((NNZ,), jnp.int32),
          pltpu.VMEM((NNZ,), jnp.int32),
          pltpu.VMEM((rpw,), jnp.int32),
          pltpu.VMEM((rpw,), jnp.int32),
          pltpu.VMEM((rpw,), jnp.float32),
          pltpu.VMEM((rpw, FD), jnp.float32),
          pltpu.SemaphoreType.DMA,
      ] + [pltpu.VMEM((_CHUNK,), jnp.int32) for _ in range(rpw // _CHUNK)],
      compiler_params=pltpu.CompilerParams(
          needs_layout_passes=False, use_tc_tiling_on_sc=False),
      interpret=interpret)


def _make_tc_kernel(B, FD, SD, OD, blk, interpret=False):
  def tc_body(f, g, a, mg, ma, w, b, o):
    wv = w[:]
    s = g[:] * mg[:] + a[:] * ma[:]
    acc = jnp.dot(f[:], wv[:FD, :], preferred_element_type=jnp.float32)
    acc = acc + jnp.dot(s, wv[FD:, :], preferred_element_type=jnp.float32)
    o[:] = acc + b[:]

  return pl.pallas_call(
      tc_body,
      grid=(B // blk,),
      in_specs=[
          pl.BlockSpec((blk, FD), lambda i: (i, 0)),
          pl.BlockSpec((blk, SD), lambda i: (i, 0)),
          pl.BlockSpec((blk, SD), lambda i: (i, 0)),
          pl.BlockSpec((blk, 1), lambda i: (i, 0)),
          pl.BlockSpec((blk, 1), lambda i: (i, 0)),
          pl.BlockSpec((FD + SD, OD), lambda i: (0, 0)),
          pl.BlockSpec((1, OD), lambda i: (0, 0)),
      ],
      out_specs=pl.BlockSpec((blk, OD), lambda i: (i, 0)),
      out_shape=jax.ShapeDtypeStruct((B, OD), jnp.float32),
      interpret=interpret)


def _run(fixed_features, geom_index, geom_value, attr_index, attr_value,
         fixed_table, geom_table, attr_table, W, b, interpret=False):
  B = fixed_features.shape[0]
  NNZ = geom_index.shape[0]
  FD = fixed_table.shape[1]
  SD = geom_table.shape[1]
  OD = W.shape[1]
  ff = fixed_features.astype(jnp.int32)
  gi = geom_index.astype(jnp.int32)
  gv = geom_value.astype(jnp.int32)
  ai = attr_index.astype(jnp.int32)
  av = attr_value.astype(jnp.int32)
  f_emb, g_rows, a_rows, mg, ma = _make_sc_kernel(B, NNZ, FD, SD, interpret)(
      ff, gi, gv, ai, av, fixed_table, geom_table, attr_table)
  return _make_tc_kernel(B, FD, SD, OD, min(2048, B), interpret)(
      f_emb, g_rows, a_rows, mg.reshape(B, 1), ma.reshape(B, 1),
      W, b.reshape(1, OD))


def kernel(fixed_features, geom_index, geom_value, attr_index, attr_value,
           fixed_table, geom_table, attr_table, W, b):
  return _run(fixed_features, geom_index, geom_value, attr_index, attr_value,
              fixed_table, geom_table, attr_table, W, b)


# speedup vs baseline: 2.3515x; 1.5593x over previous
import jax
import jax.numpy as jnp
from jax import lax
from jax.experimental import pallas as pl
from jax.experimental.pallas import tpu as pltpu
from jax.experimental.pallas import tpu_sc as plsc

_LANES = 16
_NC = 2
_NS = 16
_CHUNK = 128
_NB = 2048


def _fold_half(v):
  return ((v + _NB - 1) // _NB + 1) // 2 * _NB


def _make_fold_kernel(V, D, interpret=False):
  vh = _fold_half(V)
  n_lo = vh // _NB
  last = (V - 1) // _NB

  def body(x_lo, x_hi, o):
    o[:, :D] = x_lo[:].T
    o[:, D:] = x_hi[:].T

  return pl.pallas_call(
      body,
      grid=(n_lo,),
      in_specs=[pl.BlockSpec((D, _NB), lambda i: (0, i)),
                pl.BlockSpec((D, _NB),
                             lambda i: (0, jnp.minimum(n_lo + i, last)))],
      out_specs=pl.BlockSpec((_NB, 2 * D), lambda i: (i, 0)),
      out_shape=jax.ShapeDtypeStruct((vh, 2 * D), jnp.float32),
      interpret=interpret)


def _make_sc_kernel(B, NNZ, D2, vhf, vhs, interpret=False):
  n_workers = _NC * _NS
  rpw = B // n_workers
  assert B % n_workers == 0 and rpw % _CHUNK == 0 and NNZ % _LANES == 0

  mesh = plsc.VectorSubcoreMesh(
      core_axis_name="c", subcore_axis_name="s",
      num_cores=_NC, num_subcores=_NS)

  n_chunks = rpw // _CHUNK

  def sc_body(ff, gi, gv, ai, av, ftab, gtab, atab,
              f_out, g_out, a_out, hf_out, mg_out, ma_out,
              sidx, sval, valbuf, srcbuf, mbuf, hbuf, rows, sem, *gidx):
    wid = lax.axis_index("s") * _NC + lax.axis_index("c")
    base = wid * rpw

    def gather_rows(tab):
      descs = [
          pltpu.async_copy(tab.at[gidx[k]],
                           rows.at[pl.ds(k * _CHUNK, _CHUNK)], sem)
          for k in range(n_chunks)
      ]
      for d in descs:
        d.wait()

    with jax.named_scope("fixed_gather"):
      pltpu.sync_copy(ff.at[pl.ds(base, rpw)], sidx.at[pl.ds(0, rpw)])
      for i in range(rpw // _LANES):
        r = sidx[pl.ds(i * _LANES, _LANES)]
        hi = r >= vhf
        lo = (i * _LANES) % _CHUNK
        gidx[(i * _LANES) // _CHUNK][pl.ds(lo, _LANES)] = jnp.where(
            hi, r - vhf, r)
        hbuf[pl.ds(i * _LANES, _LANES)] = hi.astype(jnp.float32)
      gather_rows(ftab)
      pltpu.sync_copy(rows, f_out.at[pl.ds(base, rpw)])
      pltpu.sync_copy(hbuf, hf_out.at[pl.ds(base, rpw)])

    def zbody(i, _):
      srcbuf[pl.ds(i * _LANES, _LANES)] = jnp.zeros((_LANES,), jnp.int32)
      return 0
    lax.fori_loop(0, rpw // _LANES, zbody, 0)

    with jax.named_scope("winner_resolve"):
      for ih, vh_, code in ((gi, gv, 1), (ai, av, 2)):
        pltpu.sync_copy(ih, sidx)
        pltpu.sync_copy(vh_, sval)
        code16 = jnp.full((_LANES,), code, jnp.int32)

        def p1body(i, _, code16=code16):
          idx16 = sidx[pl.ds(i * _LANES, _LANES)]
          val16 = sval[pl.ds(i * _LANES, _LANES)]
          inb = (idx16 >= base) & (idx16 < base + rpw)
          local = jnp.where(inb, idx16 - base, 0)
          _, win = plsc.scan_count(local, mask=inb)
          plsc.store_scatter(valbuf, [local], val16, mask=win)
          plsc.store_scatter(srcbuf, [local], code16, mask=win)
          return 0
        lax.fori_loop(0, NNZ // _LANES, p1body, 0)

    for tab, out_hbm, m_out, code in ((gtab, g_out, mg_out, 1),
                                      (atab, a_out, ma_out, 2)):
      code16 = jnp.full((_LANES,), code, jnp.int32)

      with jax.named_scope(f"sparse_gather_{code}"):
        lane16 = lax.iota(jnp.int32, _LANES)
        for i in range(rpw // _LANES):
          v = valbuf[pl.ds(i * _LANES, _LANES)]
          s = srcbuf[pl.ds(i * _LANES, _LANES)]
          sel = s == code16
          vg = jnp.where(v >= vhs, v - vhs, v)
          lo = (i * _LANES) % _CHUNK
          gidx[(i * _LANES) // _CHUNK][pl.ds(lo, _LANES)] = jnp.where(
              sel, vg, base + lane16 + i * _LANES)
          mbuf[pl.ds(i * _LANES, _LANES)] = jnp.where(
              sel & (v < vhs), 1.0, 0.0) + jnp.where(
              sel & (v >= vhs), 2.0, 0.0)
        gather_rows(tab)
        pltpu.sync_copy(rows, out_hbm.at[pl.ds(base, rpw)])
        pltpu.sync_copy(mbuf, m_out.at[pl.ds(base, rpw)])

  return pl.kernel(
      sc_body,
      out_type=[
          jax.ShapeDtypeStruct((B, D2), jnp.float32),
          jax.ShapeDtypeStruct((B, D2), jnp.float32),
          jax.ShapeDtypeStruct((B, D2), jnp.float32),
          jax.ShapeDtypeStruct((B,), jnp.float32),
          jax.ShapeDtypeStruct((B,), jnp.float32),
          jax.ShapeDtypeStruct((B,), jnp.float32),
      ],
      mesh=mesh,
      scratch_types=[
          pltpu.VMEM((NNZ,), jnp.int32),
          pltpu.VMEM((NNZ,), jnp.int32),
          pltpu.VMEM((rpw,), jnp.int32),
          pltpu.VMEM((rpw,), jnp.int32),
          pltpu.VMEM((rpw,), jnp.float32),
          pltpu.VMEM((rpw,), jnp.float32),
          pltpu.VMEM((rpw, D2), jnp.float32),
          pltpu.SemaphoreType.DMA,
      ] + [pltpu.VMEM((_CHUNK,), jnp.int32) for _ in range(rpw // _CHUNK)],
      compiler_params=pltpu.CompilerParams(
          needs_layout_passes=False, use_tc_tiling_on_sc=False),
      interpret=interpret)


def _make_tc_kernel(B, D, OD, blk, interpret=False):
  def tc_body(f2, g2, a2, hf, mg, ma, w, b, o):
    wv = w[:]
    hfv = hf[:]
    f = jnp.where(hfv > 0.5, f2[:, D:], f2[:, :D])
    mgv, mav = mg[:], ma[:]
    zero = jnp.zeros_like(mgv)
    g = (jnp.where(mgv == 1.0, g2[:, :D], zero)
         + jnp.where(mgv == 2.0, g2[:, D:], zero))
    a = (jnp.where(mav == 1.0, a2[:, :D], zero)
         + jnp.where(mav == 2.0, a2[:, D:], zero))
    s = g + a
    acc = jnp.dot(f, wv[:D, :], preferred_element_type=jnp.float32)
    acc = acc + jnp.dot(s, wv[D:, :], preferred_element_type=jnp.float32)
    o[:] = acc + b[:]

  return pl.pallas_call(
      tc_body,
      grid=(B // blk,),
      in_specs=[
          pl.BlockSpec((blk, 2 * D), lambda i: (i, 0)),
          pl.BlockSpec((blk, 2 * D), lambda i: (i, 0)),
          pl.BlockSpec((blk, 2 * D), lambda i: (i, 0)),
          pl.BlockSpec((blk, 1), lambda i: (i, 0)),
          pl.BlockSpec((blk, 1), lambda i: (i, 0)),
          pl.BlockSpec((blk, 1), lambda i: (i, 0)),
          pl.BlockSpec((2 * D, OD), lambda i: (0, 0)),
          pl.BlockSpec((1, OD), lambda i: (0, 0)),
      ],
      out_specs=pl.BlockSpec((blk, OD), lambda i: (i, 0)),
      out_shape=jax.ShapeDtypeStruct((B, OD), jnp.float32),
      interpret=interpret)


def _run(fixed_features, geom_index, geom_value, attr_index, attr_value,
         fixed_table, geom_table, attr_table, W, b, interpret=False):
  B = fixed_features.shape[0]
  NNZ = geom_index.shape[0]
  FV, D = fixed_table.shape
  SV = geom_table.shape[0]
  OD = W.shape[1]
  ff = fixed_features.astype(jnp.int32)
  gi = geom_index.astype(jnp.int32)
  gv = geom_value.astype(jnp.int32)
  ai = attr_index.astype(jnp.int32)
  av = attr_value.astype(jnp.int32)

  fold_f = _make_fold_kernel(FV, D, interpret)
  fold_s = _make_fold_kernel(SV, D, interpret)
  ftab2 = fold_f(fixed_table.T, fixed_table.T)
  gtab2 = fold_s(geom_table.T, geom_table.T)
  atab2 = fold_s(attr_table.T, attr_table.T)
  vhf = _fold_half(FV)
  vhs = _fold_half(SV)

  f2, g2, a2, hf, mg, ma = _make_sc_kernel(B, NNZ, 2 * D, vhf, vhs,
                                           interpret)(
      ff, gi, gv, ai, av, ftab2, gtab2, atab2)
  return _make_tc_kernel(B, D, OD, min(2048, B), interpret)(
      f2, g2, a2, hf.reshape(B, 1), mg.reshape(B, 1), ma.reshape(B, 1),
      W, b.reshape(1, OD))


def kernel(fixed_features, geom_index, geom_value, attr_index, attr_value,
           fixed_table, geom_table, attr_table, W, b):
  return _run(fixed_features, geom_index, geom_value, attr_index, attr_value,
              fixed_table, geom_table, attr_table, W, b)


# speedup vs baseline: 2.3732x; 1.0093x over previous
import jax
import jax.numpy as jnp
from jax import lax
from jax.experimental import pallas as pl
from jax.experimental.pallas import tpu as pltpu
from jax.experimental.pallas import tpu_sc as plsc

_LANES = 16
_NC = 2
_NS = 16
_CHUNK = 128
_NB = 2048


def _fold_half(v):
  return ((v + _NB - 1) // _NB + 1) // 2 * _NB


def _make_fold_kernel(V, D, interpret=False):
  vh = _fold_half(V)
  n_lo = vh // _NB
  last = (V - 1) // _NB

  def body(x_lo, x_hi, o):
    eye = jnp.eye(D, dtype=jnp.float32)
    dn = (((0,), (0,)), ((), ()))
    o[:, :D] = lax.dot_general(x_lo[:], eye, dn,
                               preferred_element_type=jnp.float32)
    o[:, D:] = lax.dot_general(x_hi[:], eye, dn,
                               preferred_element_type=jnp.float32)

  return pl.pallas_call(
      body,
      grid=(n_lo,),
      in_specs=[pl.BlockSpec((D, _NB), lambda i: (0, i)),
                pl.BlockSpec((D, _NB),
                             lambda i: (0, jnp.minimum(n_lo + i, last)))],
      out_specs=pl.BlockSpec((_NB, 2 * D), lambda i: (i, 0)),
      out_shape=jax.ShapeDtypeStruct((vh, 2 * D), jnp.float32),
      interpret=interpret)


def _make_sc_sparse(B, NNZ, D2, vhs, interpret=False):
  n_workers = _NC * _NS
  rpw = B // n_workers
  assert B % n_workers == 0 and rpw % _CHUNK == 0 and NNZ % _LANES == 0

  mesh = plsc.VectorSubcoreMesh(
      core_axis_name="c", subcore_axis_name="s",
      num_cores=_NC, num_subcores=_NS)

  n_chunks = rpw // _CHUNK

  def sc_body(gi, gv, ai, av, gtab, atab,
              g_out, a_out, mg_out, ma_out,
              sidx, sval, valbuf, srcbuf, mbuf, rows, sem, *gidx):
    wid = lax.axis_index("s") * _NC + lax.axis_index("c")
    base = wid * rpw

    def gather_rows(tab):
      descs = [
          pltpu.async_copy(tab.at[gidx[k]],
                           rows.at[pl.ds(k * _CHUNK, _CHUNK)], sem)
          for k in range(n_chunks)
      ]
      for d in descs:
        d.wait()

    def zbody(i, _):
      srcbuf[pl.ds(i * _LANES, _LANES)] = jnp.zeros((_LANES,), jnp.int32)
      return 0
    lax.fori_loop(0, rpw // _LANES, zbody, 0)

    with jax.named_scope("winner_resolve"):
      for ih, vh_, code in ((gi, gv, 1), (ai, av, 2)):
        pltpu.sync_copy(ih, sidx)
        pltpu.sync_copy(vh_, sval)
        code16 = jnp.full((_LANES,), code, jnp.int32)

        def p1body(i, _, code16=code16):
          idx16 = sidx[pl.ds(i * _LANES, _LANES)]
          val16 = sval[pl.ds(i * _LANES, _LANES)]
          inb = (idx16 >= base) & (idx16 < base + rpw)
          local = jnp.where(inb, idx16 - base, 0)
          _, win = plsc.scan_count(local, mask=inb)
          plsc.store_scatter(valbuf, [local], val16, mask=win)
          plsc.store_scatter(srcbuf, [local], code16, mask=win)
          return 0
        lax.fori_loop(0, NNZ // _LANES, p1body, 0)

    for tab, out_hbm, m_out, code in ((gtab, g_out, mg_out, 1),
                                      (atab, a_out, ma_out, 2)):
      code16 = jnp.full((_LANES,), code, jnp.int32)

      with jax.named_scope(f"sparse_gather_{code}"):
        lane16 = lax.iota(jnp.int32, _LANES)
        for i in range(rpw // _LANES):
          v = valbuf[pl.ds(i * _LANES, _LANES)]
          s = srcbuf[pl.ds(i * _LANES, _LANES)]
          sel = s == code16
          vg = jnp.where(v >= vhs, v - vhs, v)
          lo = (i * _LANES) % _CHUNK
          gidx[(i * _LANES) // _CHUNK][pl.ds(lo, _LANES)] = jnp.where(
              sel, vg, base + lane16 + i * _LANES)
          mbuf[pl.ds(i * _LANES, _LANES)] = jnp.where(
              sel & (v < vhs), 1.0, 0.0) + jnp.where(
              sel & (v >= vhs), 2.0, 0.0)
        gather_rows(tab)
        pltpu.sync_copy(rows, out_hbm.at[pl.ds(base, rpw)])
        pltpu.sync_copy(mbuf, m_out.at[pl.ds(base, rpw)])

  return pl.kernel(
      sc_body,
      out_type=[
          jax.ShapeDtypeStruct((B, D2), jnp.float32),
          jax.ShapeDtypeStruct((B, D2), jnp.float32),
          jax.ShapeDtypeStruct((B,), jnp.float32),
          jax.ShapeDtypeStruct((B,), jnp.float32),
      ],
      mesh=mesh,
      scratch_types=[
          pltpu.VMEM((NNZ,), jnp.int32),
          pltpu.VMEM((NNZ,), jnp.int32),
          pltpu.VMEM((rpw,), jnp.int32),
          pltpu.VMEM((rpw,), jnp.int32),
          pltpu.VMEM((rpw,), jnp.float32),
          pltpu.VMEM((rpw, D2), jnp.float32),
          pltpu.SemaphoreType.DMA,
      ] + [pltpu.VMEM((_CHUNK,), jnp.int32) for _ in range(rpw // _CHUNK)],
      compiler_params=pltpu.CompilerParams(
          needs_layout_passes=False, use_tc_tiling_on_sc=False),
      interpret=interpret)


def _make_sc_fixed(B, D2, vhf, interpret=False):
  n_workers = _NC * _NS
  rpw = B // n_workers
  mesh = plsc.VectorSubcoreMesh(
      core_axis_name="c", subcore_axis_name="s",
      num_cores=_NC, num_subcores=_NS)
  n_chunks = rpw // _CHUNK

  def sc_body(ff, ftab, f_out, hf_out, fidx, hbuf, rows, sem, *gidx):
    wid = lax.axis_index("s") * _NC + lax.axis_index("c")
    base = wid * rpw

    with jax.named_scope("fixed_gather"):
      pltpu.sync_copy(ff.at[pl.ds(base, rpw)], fidx)
      for i in range(rpw // _LANES):
        r = fidx[pl.ds(i * _LANES, _LANES)]
        hi = r >= vhf
        lo = (i * _LANES) % _CHUNK
        gidx[(i * _LANES) // _CHUNK][pl.ds(lo, _LANES)] = jnp.where(
            hi, r - vhf, r)
        hbuf[pl.ds(i * _LANES, _LANES)] = hi.astype(jnp.float32)
      descs = [
          pltpu.async_copy(ftab.at[gidx[k]],
                           rows.at[pl.ds(k * _CHUNK, _CHUNK)], sem)
          for k in range(n_chunks)
      ]
      for d in descs:
        d.wait()
      pltpu.sync_copy(rows, f_out.at[pl.ds(base, rpw)])
      pltpu.sync_copy(hbuf, hf_out.at[pl.ds(base, rpw)])

  return pl.kernel(
      sc_body,
      out_type=[
          jax.ShapeDtypeStruct((B, D2), jnp.float32),
          jax.ShapeDtypeStruct((B,), jnp.float32),
      ],
      mesh=mesh,
      scratch_types=[
          pltpu.VMEM((rpw,), jnp.int32),
          pltpu.VMEM((rpw,), jnp.float32),
          pltpu.VMEM((rpw, D2), jnp.float32),
          pltpu.SemaphoreType.DMA,
      ] + [pltpu.VMEM((_CHUNK,), jnp.int32) for _ in range(rpw // _CHUNK)],
      compiler_params=pltpu.CompilerParams(
          needs_layout_passes=False, use_tc_tiling_on_sc=False),
      interpret=interpret)


def _make_tc_kernel(B, D, OD, blk, interpret=False):
  def tc_body(f2, g2, a2, hf, mg, ma, w, b, o):
    wv = w[:]
    hfv = hf[:]
    f = jnp.where(hfv > 0.5, f2[:, D:], f2[:, :D])
    mgv, mav = mg[:], ma[:]
    zero = jnp.zeros_like(mgv)
    g = (jnp.where(mgv == 1.0, g2[:, :D], zero)
         + jnp.where(mgv == 2.0, g2[:, D:], zero))
    a = (jnp.where(mav == 1.0, a2[:, :D], zero)
         + jnp.where(mav == 2.0, a2[:, D:], zero))
    s = g + a
    acc = jnp.dot(f, wv[:D, :], preferred_element_type=jnp.float32)
    acc = acc + jnp.dot(s, wv[D:, :], preferred_element_type=jnp.float32)
    o[:] = acc + b[:]

  return pl.pallas_call(
      tc_body,
      grid=(B // blk,),
      in_specs=[
          pl.BlockSpec((blk, 2 * D), lambda i: (i, 0)),
          pl.BlockSpec((blk, 2 * D), lambda i: (i, 0)),
          pl.BlockSpec((blk, 2 * D), lambda i: (i, 0)),
          pl.BlockSpec((blk, 1), lambda i: (i, 0)),
          pl.BlockSpec((blk, 1), lambda i: (i, 0)),
          pl.BlockSpec((blk, 1), lambda i: (i, 0)),
          pl.BlockSpec((2 * D, OD), lambda i: (0, 0)),
          pl.BlockSpec((1, OD), lambda i: (0, 0)),
      ],
      out_specs=pl.BlockSpec((blk, OD), lambda i: (i, 0)),
      out_shape=jax.ShapeDtypeStruct((B, OD), jnp.float32),
      interpret=interpret)


def _run(fixed_features, geom_index, geom_value, attr_index, attr_value,
         fixed_table, geom_table, attr_table, W, b, interpret=False):
  B = fixed_features.shape[0]
  NNZ = geom_index.shape[0]
  FV, D = fixed_table.shape
  SV = geom_table.shape[0]
  OD = W.shape[1]
  ff = fixed_features.astype(jnp.int32)
  gi = geom_index.astype(jnp.int32)
  gv = geom_value.astype(jnp.int32)
  ai = attr_index.astype(jnp.int32)
  av = attr_value.astype(jnp.int32)

  fold_f = _make_fold_kernel(FV, D, interpret)
  fold_s = _make_fold_kernel(SV, D, interpret)
  gtab2 = fold_s(geom_table.T, geom_table.T)
  atab2 = fold_s(attr_table.T, attr_table.T)
  ftab2 = fold_f(fixed_table.T, fixed_table.T)
  vhf = _fold_half(FV)
  vhs = _fold_half(SV)

  g2, a2, mg, ma = _make_sc_sparse(B, NNZ, 2 * D, vhs, interpret)(
      gi, gv, ai, av, gtab2, atab2)
  f2, hf = _make_sc_fixed(B, 2 * D, vhf, interpret)(ff, ftab2)
  return _make_tc_kernel(B, D, OD, min(2048, B), interpret)(
      f2, g2, a2, hf.reshape(B, 1), mg.reshape(B, 1), ma.reshape(B, 1),
      W, b.reshape(1, OD))


def kernel(fixed_features, geom_index, geom_value, attr_index, attr_value,
           fixed_table, geom_table, attr_table, W, b):
  return _run(fixed_features, geom_index, geom_value, attr_index, attr_value,
              fixed_table, geom_table, attr_table, W, b)


# speedup vs baseline: 3.0769x; 1.2965x over previous
import jax
import jax.numpy as jnp
from jax import lax
from jax.experimental import pallas as pl
from jax.experimental.pallas import tpu as pltpu
from jax.experimental.pallas import tpu_sc as plsc

_LANES = 16
_NC = 2
_NS = 16
_CHUNK = 128
_NB = 8192


def _fold_half(v):
  return ((v + _NB - 1) // _NB + 1) // 2 * _NB


def _make_fold_kernel(V, D, interpret=False):
  vh = _fold_half(V)
  n_lo = vh // _NB
  last = (V - 1) // _NB

  def body(x_lo, x_hi, o):
    eye = jnp.eye(D, dtype=jnp.float32)
    dn = (((0,), (0,)), ((), ()))
    o[:, :D] = lax.dot_general(x_lo[:], eye, dn,
                               preferred_element_type=jnp.float32)
    o[:, D:] = lax.dot_general(x_hi[:], eye, dn,
                               preferred_element_type=jnp.float32)

  return pl.pallas_call(
      body,
      grid=(n_lo,),
      in_specs=[pl.BlockSpec((D, _NB), lambda i: (0, i)),
                pl.BlockSpec((D, _NB),
                             lambda i: (0, jnp.minimum(n_lo + i, last)))],
      out_specs=pl.BlockSpec((_NB, 2 * D), lambda i: (i, 0)),
      out_shape=jax.ShapeDtypeStruct((vh, 2 * D), jnp.float32),
      interpret=interpret)


def _make_sc_sparse(B, NNZ, D2, vhs, interpret=False):
  n_workers = _NC * _NS
  rpw = B // n_workers
  assert B % n_workers == 0 and rpw % _CHUNK == 0 and NNZ % _LANES == 0

  mesh = plsc.VectorSubcoreMesh(
      core_axis_name="c", subcore_axis_name="s",
      num_cores=_NC, num_subcores=_NS)

  n_chunks = rpw // _CHUNK

  def sc_body(gi, gv, ai, av, gtab, atab,
              g_out, a_out, mg_out, ma_out,
              sidx, sval, valbuf, srcbuf, mbuf, rows, sem, *gidx):
    wid = lax.axis_index("s") * _NC + lax.axis_index("c")
    base = wid * rpw

    def gather_rows(tab):
      descs = [
          pltpu.async_copy(tab.at[gidx[k]],
                           rows.at[pl.ds(k * _CHUNK, _CHUNK)], sem)
          for k in range(n_chunks)
      ]
      for d in descs:
        d.wait()

    def zbody(i, _):
      srcbuf[pl.ds(i * _LANES, _LANES)] = jnp.zeros((_LANES,), jnp.int32)
      return 0
    lax.fori_loop(0, rpw // _LANES, zbody, 0)

    with jax.named_scope("winner_resolve"):
      for ih, vh_, code in ((gi, gv, 1), (ai, av, 2)):
        pltpu.sync_copy(ih, sidx)
        pltpu.sync_copy(vh_, sval)
        code16 = jnp.full((_LANES,), code, jnp.int32)

        def p1body(i, _, code16=code16):
          idx16 = sidx[pl.ds(i * _LANES, _LANES)]
          val16 = sval[pl.ds(i * _LANES, _LANES)]
          inb = (idx16 >= base) & (idx16 < base + rpw)
          local = jnp.where(inb, idx16 - base, 0)
          _, win = plsc.scan_count(local, mask=inb)
          plsc.store_scatter(valbuf, [local], val16, mask=win)
          plsc.store_scatter(srcbuf, [local], code16, mask=win)
          return 0
        lax.fori_loop(0, NNZ // _LANES, p1body, 0)

    for tab, out_hbm, m_out, code in ((gtab, g_out, mg_out, 1),
                                      (atab, a_out, ma_out, 2)):
      code16 = jnp.full((_LANES,), code, jnp.int32)

      with jax.named_scope(f"sparse_gather_{code}"):
        lane16 = lax.iota(jnp.int32, _LANES)
        for i in range(rpw // _LANES):
          v = valbuf[pl.ds(i * _LANES, _LANES)]
          s = srcbuf[pl.ds(i * _LANES, _LANES)]
          sel = s == code16
          vg = jnp.where(v >= vhs, v - vhs, v)
          lo = (i * _LANES) % _CHUNK
          gidx[(i * _LANES) // _CHUNK][pl.ds(lo, _LANES)] = jnp.where(
              sel, vg, base + lane16 + i * _LANES)
          mbuf[pl.ds(i * _LANES, _LANES)] = jnp.where(
              sel & (v < vhs), 1.0, 0.0) + jnp.where(
              sel & (v >= vhs), 2.0, 0.0)
        gather_rows(tab)
        pltpu.sync_copy(rows, out_hbm.at[pl.ds(base, rpw)])
        pltpu.sync_copy(mbuf, m_out.at[pl.ds(base, rpw)])

  return pl.kernel(
      sc_body,
      out_type=[
          jax.ShapeDtypeStruct((B, D2), jnp.float32),
          jax.ShapeDtypeStruct((B, D2), jnp.float32),
          jax.ShapeDtypeStruct((B,), jnp.float32),
          jax.ShapeDtypeStruct((B,), jnp.float32),
      ],
      mesh=mesh,
      scratch_types=[
          pltpu.VMEM((NNZ,), jnp.int32),
          pltpu.VMEM((NNZ,), jnp.int32),
          pltpu.VMEM((rpw,), jnp.int32),
          pltpu.VMEM((rpw,), jnp.int32),
          pltpu.VMEM((rpw,), jnp.float32),
          pltpu.VMEM((rpw, D2), jnp.float32),
          pltpu.SemaphoreType.DMA,
      ] + [pltpu.VMEM((_CHUNK,), jnp.int32) for _ in range(rpw // _CHUNK)],
      compiler_params=pltpu.CompilerParams(
          needs_layout_passes=False, use_tc_tiling_on_sc=False),
      interpret=interpret)


def _make_sc_fixed(B, D2, vhf, interpret=False):
  n_workers = _NC * _NS
  rpw = B // n_workers
  mesh = plsc.VectorSubcoreMesh(
      core_axis_name="c", subcore_axis_name="s",
      num_cores=_NC, num_subcores=_NS)
  n_chunks = rpw // _CHUNK

  def sc_body(ff, ftab, f_out, hf_out, fidx, hbuf, rows, sem, *gidx):
    wid = lax.axis_index("s") * _NC + lax.axis_index("c")
    base = wid * rpw

    with jax.named_scope("fixed_gather"):
      pltpu.sync_copy(ff.at[pl.ds(base, rpw)], fidx)
      for i in range(rpw // _LANES):
        r = fidx[pl.ds(i * _LANES, _LANES)]
        hi = r >= vhf
        lo = (i * _LANES) % _CHUNK
        gidx[(i * _LANES) // _CHUNK][pl.ds(lo, _LANES)] = jnp.where(
            hi, r - vhf, r)
        hbuf[pl.ds(i * _LANES, _LANES)] = hi.astype(jnp.float32)
      descs = [
          pltpu.async_copy(ftab.at[gidx[k]],
                           rows.at[pl.ds(k * _CHUNK, _CHUNK)], sem)
          for k in range(n_chunks)
      ]
      for d in descs:
        d.wait()
      pltpu.sync_copy(rows, f_out.at[pl.ds(base, rpw)])
      pltpu.sync_copy(hbuf, hf_out.at[pl.ds(base, rpw)])

  return pl.kernel(
      sc_body,
      out_type=[
          jax.ShapeDtypeStruct((B, D2), jnp.float32),
          jax.ShapeDtypeStruct((B,), jnp.float32),
      ],
      mesh=mesh,
      scratch_types=[
          pltpu.VMEM((rpw,), jnp.int32),
          pltpu.VMEM((rpw,), jnp.float32),
          pltpu.VMEM((rpw, D2), jnp.float32),
          pltpu.SemaphoreType.DMA,
      ] + [pltpu.VMEM((_CHUNK,), jnp.int32) for _ in range(rpw // _CHUNK)],
      compiler_params=pltpu.CompilerParams(
          needs_layout_passes=False, use_tc_tiling_on_sc=False),
      interpret=interpret)


def _make_tc_kernel(B, D, OD, blk, interpret=False):
  def tc_body(f2, g2, a2, hf, mg, ma, w, b, o):
    wv = w[:]
    hfv = hf[:]
    f = jnp.where(hfv > 0.5, f2[:, D:], f2[:, :D])
    mgv, mav = mg[:], ma[:]
    zero = jnp.zeros_like(mgv)
    g = (jnp.where(mgv == 1.0, g2[:, :D], zero)
         + jnp.where(mgv == 2.0, g2[:, D:], zero))
    a = (jnp.where(mav == 1.0, a2[:, :D], zero)
         + jnp.where(mav == 2.0, a2[:, D:], zero))
    s = g + a
    acc = jnp.dot(f, wv[:D, :], preferred_element_type=jnp.float32)
    acc = acc + jnp.dot(s, wv[D:, :], preferred_element_type=jnp.float32)
    o[:] = acc + b[:]

  return pl.pallas_call(
      tc_body,
      grid=(B // blk,),
      in_specs=[
          pl.BlockSpec((blk, 2 * D), lambda i: (i, 0)),
          pl.BlockSpec((blk, 2 * D), lambda i: (i, 0)),
          pl.BlockSpec((blk, 2 * D), lambda i: (i, 0)),
          pl.BlockSpec((blk, 1), lambda i: (i, 0)),
          pl.BlockSpec((blk, 1), lambda i: (i, 0)),
          pl.BlockSpec((blk, 1), lambda i: (i, 0)),
          pl.BlockSpec((2 * D, OD), lambda i: (0, 0)),
          pl.BlockSpec((1, OD), lambda i: (0, 0)),
      ],
      out_specs=pl.BlockSpec((blk, OD), lambda i: (i, 0)),
      out_shape=jax.ShapeDtypeStruct((B, OD), jnp.float32),
      interpret=interpret)


def _run(fixed_features, geom_index, geom_value, attr_index, attr_value,
         fixed_table, geom_table, attr_table, W, b, interpret=False):
  B = fixed_features.shape[0]
  NNZ = geom_index.shape[0]
  FV, D = fixed_table.shape
  SV = geom_table.shape[0]
  OD = W.shape[1]
  ff = fixed_features.astype(jnp.int32)
  gi = geom_index.astype(jnp.int32)
  gv = geom_value.astype(jnp.int32)
  ai = attr_index.astype(jnp.int32)
  av = attr_value.astype(jnp.int32)

  fold_f = _make_fold_kernel(FV, D, interpret)
  fold_s = _make_fold_kernel(SV, D, interpret)
  gtab2 = fold_s(geom_table.T, geom_table.T)
  atab2 = fold_s(attr_table.T, attr_table.T)
  ftab2 = fold_f(fixed_table.T, fixed_table.T)
  vhf = _fold_half(FV)
  vhs = _fold_half(SV)

  g2, a2, mg, ma = _make_sc_sparse(B, NNZ, 2 * D, vhs, interpret)(
      gi, gv, ai, av, gtab2, atab2)
  f2, hf = _make_sc_fixed(B, 2 * D, vhf, interpret)(ff, ftab2)
  return _make_tc_kernel(B, D, OD, min(2048, B), interpret)(
      f2, g2, a2, hf.reshape(B, 1), mg.reshape(B, 1), ma.reshape(B, 1),
      W, b.reshape(1, OD))


def kernel(fixed_features, geom_index, geom_value, attr_index, attr_value,
           fixed_table, geom_table, attr_table, W, b):
  return _run(fixed_features, geom_index, geom_value, attr_index, attr_value,
              fixed_table, geom_table, attr_table, W, b)


# speedup vs baseline: 3.1134x; 1.0118x over previous
import jax
import jax.numpy as jnp
from jax import lax
from jax.experimental import pallas as pl
from jax.experimental.pallas import tpu as pltpu
from jax.experimental.pallas import tpu_sc as plsc

_LANES = 16
_NC = 2
_NS = 16
_CHUNK = 128
_NB = 16384


def _fold_half(v):
  return ((v + _NB - 1) // _NB + 1) // 2 * _NB


def _make_fold_kernel(V, D, interpret=False):
  vh = _fold_half(V)
  n_lo = vh // _NB
  last = (V - 1) // _NB

  def body(x_lo, x_hi, o):
    eye = jnp.eye(D, dtype=jnp.float32)
    dn = (((0,), (0,)), ((), ()))
    o[:, :D] = lax.dot_general(x_lo[:], eye, dn,
                               preferred_element_type=jnp.float32)
    o[:, D:] = lax.dot_general(x_hi[:], eye, dn,
                               preferred_element_type=jnp.float32)

  return pl.pallas_call(
      body,
      grid=(n_lo,),
      in_specs=[pl.BlockSpec((D, _NB), lambda i: (0, i)),
                pl.BlockSpec((D, _NB),
                             lambda i: (0, jnp.minimum(n_lo + i, last)))],
      out_specs=pl.BlockSpec((_NB, 2 * D), lambda i: (i, 0)),
      out_shape=jax.ShapeDtypeStruct((vh, 2 * D), jnp.float32),
      interpret=interpret)


def _make_sc_sparse(B, NNZ, D2, vhs, interpret=False):
  n_workers = _NC * _NS
  rpw = B // n_workers
  assert B % n_workers == 0 and rpw % _CHUNK == 0 and NNZ % _LANES == 0

  mesh = plsc.VectorSubcoreMesh(
      core_axis_name="c", subcore_axis_name="s",
      num_cores=_NC, num_subcores=_NS)

  n_chunks = rpw // _CHUNK

  def sc_body(gi, gv, ai, av, gtab, atab,
              g_out, a_out, mg_out, ma_out,
              sidx, sval, valbuf, srcbuf, mbuf, rows, sem, *gidx):
    wid = lax.axis_index("s") * _NC + lax.axis_index("c")
    base = wid * rpw

    def gather_rows(tab):
      descs = [
          pltpu.async_copy(tab.at[gidx[k]],
                           rows.at[pl.ds(k * _CHUNK, _CHUNK)], sem)
          for k in range(n_chunks)
      ]
      for d in descs:
        d.wait()

    def zbody(i, _):
      srcbuf[pl.ds(i * _LANES, _LANES)] = jnp.zeros((_LANES,), jnp.int32)
      return 0
    lax.fori_loop(0, rpw // _LANES, zbody, 0)

    with jax.named_scope("winner_resolve"):
      for ih, vh_, code in ((gi, gv, 1), (ai, av, 2)):
        pltpu.sync_copy(ih, sidx)
        pltpu.sync_copy(vh_, sval)
        code16 = jnp.full((_LANES,), code, jnp.int32)

        def p1body(i, _, code16=code16):
          idx16 = sidx[pl.ds(i * _LANES, _LANES)]
          val16 = sval[pl.ds(i * _LANES, _LANES)]
          inb = (idx16 >= base) & (idx16 < base + rpw)
          local = jnp.where(inb, idx16 - base, 0)
          _, win = plsc.scan_count(local, mask=inb)
          plsc.store_scatter(valbuf, [local], val16, mask=win)
          plsc.store_scatter(srcbuf, [local], code16, mask=win)
          return 0
        lax.fori_loop(0, NNZ // _LANES, p1body, 0)

    for tab, out_hbm, m_out, code in ((gtab, g_out, mg_out, 1),
                                      (atab, a_out, ma_out, 2)):
      code16 = jnp.full((_LANES,), code, jnp.int32)

      with jax.named_scope(f"sparse_gather_{code}"):
        lane16 = lax.iota(jnp.int32, _LANES)
        for i in range(rpw // _LANES):
          v = valbuf[pl.ds(i * _LANES, _LANES)]
          s = srcbuf[pl.ds(i * _LANES, _LANES)]
          sel = s == code16
          vg = jnp.where(v >= vhs, v - vhs, v)
          lo = (i * _LANES) % _CHUNK
          gidx[(i * _LANES) // _CHUNK][pl.ds(lo, _LANES)] = jnp.where(
              sel, vg, base + lane16 + i * _LANES)
          mbuf[pl.ds(i * _LANES, _LANES)] = jnp.where(
              sel & (v < vhs), 1.0, 0.0) + jnp.where(
              sel & (v >= vhs), 2.0, 0.0)
        gather_rows(tab)
        pltpu.sync_copy(rows, out_hbm.at[pl.ds(base, rpw)])
        pltpu.sync_copy(mbuf, m_out.at[pl.ds(base, rpw)])

  return pl.kernel(
      sc_body,
      out_type=[
          jax.ShapeDtypeStruct((B, D2), jnp.float32),
          jax.ShapeDtypeStruct((B, D2), jnp.float32),
          jax.ShapeDtypeStruct((B,), jnp.float32),
          jax.ShapeDtypeStruct((B,), jnp.float32),
      ],
      mesh=mesh,
      scratch_types=[
          pltpu.VMEM((NNZ,), jnp.int32),
          pltpu.VMEM((NNZ,), jnp.int32),
          pltpu.VMEM((rpw,), jnp.int32),
          pltpu.VMEM((rpw,), jnp.int32),
          pltpu.VMEM((rpw,), jnp.float32),
          pltpu.VMEM((rpw, D2), jnp.float32),
          pltpu.SemaphoreType.DMA,
      ] + [pltpu.VMEM((_CHUNK,), jnp.int32) for _ in range(rpw // _CHUNK)],
      compiler_params=pltpu.CompilerParams(
          needs_layout_passes=False, use_tc_tiling_on_sc=False),
      interpret=interpret)


def _make_sc_fixed(B, D2, vhf, interpret=False):
  n_workers = _NC * _NS
  rpw = B // n_workers
  mesh = plsc.VectorSubcoreMesh(
      core_axis_name="c", subcore_axis_name="s",
      num_cores=_NC, num_subcores=_NS)
  n_chunks = rpw // _CHUNK

  def sc_body(ff, ftab, f_out, hf_out, fidx, hbuf, rows, sem, *gidx):
    wid = lax.axis_index("s") * _NC + lax.axis_index("c")
    base = wid * rpw

    with jax.named_scope("fixed_gather"):
      pltpu.sync_copy(ff.at[pl.ds(base, rpw)], fidx)
      for i in range(rpw // _LANES):
        r = fidx[pl.ds(i * _LANES, _LANES)]
        hi = r >= vhf
        lo = (i * _LANES) % _CHUNK
        gidx[(i * _LANES) // _CHUNK][pl.ds(lo, _LANES)] = jnp.where(
            hi, r - vhf, r)
        hbuf[pl.ds(i * _LANES, _LANES)] = hi.astype(jnp.float32)
      descs = [
          pltpu.async_copy(ftab.at[gidx[k]],
                           rows.at[pl.ds(k * _CHUNK, _CHUNK)], sem)
          for k in range(n_chunks)
      ]
      for d in descs:
        d.wait()
      pltpu.sync_copy(rows, f_out.at[pl.ds(base, rpw)])
      pltpu.sync_copy(hbuf, hf_out.at[pl.ds(base, rpw)])

  return pl.kernel(
      sc_body,
      out_type=[
          jax.ShapeDtypeStruct((B, D2), jnp.float32),
          jax.ShapeDtypeStruct((B,), jnp.float32),
      ],
      mesh=mesh,
      scratch_types=[
          pltpu.VMEM((rpw,), jnp.int32),
          pltpu.VMEM((rpw,), jnp.float32),
          pltpu.VMEM((rpw, D2), jnp.float32),
          pltpu.SemaphoreType.DMA,
      ] + [pltpu.VMEM((_CHUNK,), jnp.int32) for _ in range(rpw // _CHUNK)],
      compiler_params=pltpu.CompilerParams(
          needs_layout_passes=False, use_tc_tiling_on_sc=False),
      interpret=interpret)


def _make_tc_kernel(B, D, OD, blk, interpret=False):
  def tc_body(f2, g2, a2, hf, mg, ma, w, b, o):
    wv = w[:]
    hfv = hf[:]
    f = jnp.where(hfv > 0.5, f2[:, D:], f2[:, :D])
    mgv, mav = mg[:], ma[:]
    zero = jnp.zeros_like(mgv)
    g = (jnp.where(mgv == 1.0, g2[:, :D], zero)
         + jnp.where(mgv == 2.0, g2[:, D:], zero))
    a = (jnp.where(mav == 1.0, a2[:, :D], zero)
         + jnp.where(mav == 2.0, a2[:, D:], zero))
    s = g + a
    acc = jnp.dot(f, wv[:D, :], preferred_element_type=jnp.float32)
    acc = acc + jnp.dot(s, wv[D:, :], preferred_element_type=jnp.float32)
    o[:] = acc + b[:]

  return pl.pallas_call(
      tc_body,
      grid=(B // blk,),
      in_specs=[
          pl.BlockSpec((blk, 2 * D), lambda i: (i, 0)),
          pl.BlockSpec((blk, 2 * D), lambda i: (i, 0)),
          pl.BlockSpec((blk, 2 * D), lambda i: (i, 0)),
          pl.BlockSpec((blk, 1), lambda i: (i, 0)),
          pl.BlockSpec((blk, 1), lambda i: (i, 0)),
          pl.BlockSpec((blk, 1), lambda i: (i, 0)),
          pl.BlockSpec((2 * D, OD), lambda i: (0, 0)),
          pl.BlockSpec((1, OD), lambda i: (0, 0)),
      ],
      out_specs=pl.BlockSpec((blk, OD), lambda i: (i, 0)),
      out_shape=jax.ShapeDtypeStruct((B, OD), jnp.float32),
      interpret=interpret)


def _run(fixed_features, geom_index, geom_value, attr_index, attr_value,
         fixed_table, geom_table, attr_table, W, b, interpret=False):
  B = fixed_features.shape[0]
  NNZ = geom_index.shape[0]
  FV, D = fixed_table.shape
  SV = geom_table.shape[0]
  OD = W.shape[1]
  ff = fixed_features.astype(jnp.int32)
  gi = geom_index.astype(jnp.int32)
  gv = geom_value.astype(jnp.int32)
  ai = attr_index.astype(jnp.int32)
  av = attr_value.astype(jnp.int32)

  fold_f = _make_fold_kernel(FV, D, interpret)
  fold_s = _make_fold_kernel(SV, D, interpret)
  gtab2 = fold_s(geom_table.T, geom_table.T)
  atab2 = fold_s(attr_table.T, attr_table.T)
  ftab2 = fold_f(fixed_table.T, fixed_table.T)
  vhf = _fold_half(FV)
  vhs = _fold_half(SV)

  g2, a2, mg, ma = _make_sc_sparse(B, NNZ, 2 * D, vhs, interpret)(
      gi, gv, ai, av, gtab2, atab2)
  f2, hf = _make_sc_fixed(B, 2 * D, vhf, interpret)(ff, ftab2)
  return _make_tc_kernel(B, D, OD, min(2048, B), interpret)(
      f2, g2, a2, hf.reshape(B, 1), mg.reshape(B, 1), ma.reshape(B, 1),
      W, b.reshape(1, OD))


def kernel(fixed_features, geom_index, geom_value, attr_index, attr_value,
           fixed_table, geom_table, attr_table, W, b):
  return _run(fixed_features, geom_index, geom_value, attr_index, attr_value,
              fixed_table, geom_table, attr_table, W, b)


# speedup vs baseline: 3.1203x; 1.0022x over previous
import jax
import jax.numpy as jnp
from jax import lax
from jax.experimental import pallas as pl
from jax.experimental.pallas import tpu as pltpu
from jax.experimental.pallas import tpu_sc as plsc

_LANES = 16
_NC = 2
_NS = 16
_CHUNK = 128
_NB = 16384


def _fold_half(v):
  return ((v + _NB - 1) // _NB + 1) // 2 * _NB


def _make_fold_kernel(V, D, interpret=False):
  vh = _fold_half(V)
  n_lo = vh // _NB
  last = (V - 1) // _NB

  def body(x_lo, x_hi, o):
    eye = jnp.eye(D, dtype=jnp.float32)
    dn = (((0,), (0,)), ((), ()))
    o[:, :D] = lax.dot_general(x_lo[:], eye, dn,
                               preferred_element_type=jnp.float32)
    o[:, D:] = lax.dot_general(x_hi[:], eye, dn,
                               preferred_element_type=jnp.float32)

  return pl.pallas_call(
      body,
      grid=(n_lo,),
      in_specs=[pl.BlockSpec((D, _NB), lambda i: (0, i)),
                pl.BlockSpec((D, _NB),
                             lambda i: (0, jnp.minimum(n_lo + i, last)))],
      out_specs=pl.BlockSpec((_NB, 2 * D), lambda i: (i, 0)),
      out_shape=jax.ShapeDtypeStruct((vh, 2 * D), jnp.float32),
      interpret=interpret)


def _make_sc_sparse(B, NNZ, D2, vhs, interpret=False):
  n_workers = _NC * _NS
  rpw = B // n_workers
  assert B % n_workers == 0 and rpw % _CHUNK == 0 and NNZ % _LANES == 0

  mesh = plsc.VectorSubcoreMesh(
      core_axis_name="c", subcore_axis_name="s",
      num_cores=_NC, num_subcores=_NS)

  n_chunks = rpw // _CHUNK

  def sc_body(gi, gv, ai, av, gtab, atab,
              g_out, a_out, mg_out, ma_out,
              sidx, sval, valbuf, srcbuf, mbuf, rows, sem, *gidx):
    wid = lax.axis_index("s") * _NC + lax.axis_index("c")
    base = wid * rpw

    def gather_rows(tab):
      descs = [
          pltpu.async_copy(tab.at[gidx[k]],
                           rows.at[pl.ds(k * _CHUNK, _CHUNK)], sem)
          for k in range(n_chunks)
      ]
      for d in descs:
        d.wait()

    def zbody(i, _):
      srcbuf[pl.ds(i * _LANES, _LANES)] = jnp.zeros((_LANES,), jnp.int32)
      return 0
    lax.fori_loop(0, rpw // _LANES, zbody, 0)

    with jax.named_scope("winner_resolve"):
      for ih, vh_, code in ((gi, gv, 1), (ai, av, 2)):
        pltpu.sync_copy(ih, sidx)
        pltpu.sync_copy(vh_, sval)
        code16 = jnp.full((_LANES,), code, jnp.int32)

        def p1body(i, _, code16=code16):
          idx16 = sidx[pl.ds(i * _LANES, _LANES)]
          val16 = sval[pl.ds(i * _LANES, _LANES)]
          inb = (idx16 >= base) & (idx16 < base + rpw)
          local = jnp.where(inb, idx16 - base, 0)
          _, win = plsc.scan_count(local, mask=inb)
          plsc.store_scatter(valbuf, [local], val16, mask=win)
          plsc.store_scatter(srcbuf, [local], code16, mask=win)
          return 0
        lax.fori_loop(0, NNZ // _LANES, p1body, 0)

    for tab, out_hbm, m_out, code in ((gtab, g_out, mg_out, 1),
                                      (atab, a_out, ma_out, 2)):
      code16 = jnp.full((_LANES,), code, jnp.int32)

      with jax.named_scope(f"sparse_gather_{code}"):
        lane16 = lax.iota(jnp.int32, _LANES)
        for i in range(rpw // _LANES):
          v = valbuf[pl.ds(i * _LANES, _LANES)]
          s = srcbuf[pl.ds(i * _LANES, _LANES)]
          sel = s == code16
          vg = jnp.where(v >= vhs, v - vhs, v)
          lo = (i * _LANES) % _CHUNK
          gidx[(i * _LANES) // _CHUNK][pl.ds(lo, _LANES)] = jnp.where(
              sel, vg, base + lane16 + i * _LANES)
          mbuf[pl.ds(i * _LANES, _LANES)] = jnp.where(
              sel & (v < vhs), 1.0, 0.0) + jnp.where(
              sel & (v >= vhs), 2.0, 0.0)
        gather_rows(tab)
        pltpu.sync_copy(rows, out_hbm.at[pl.ds(base, rpw)])
        pltpu.sync_copy(mbuf, m_out.at[pl.ds(base, rpw)])

  return pl.kernel(
      sc_body,
      out_type=[
          jax.ShapeDtypeStruct((B, D2), jnp.float32),
          jax.ShapeDtypeStruct((B, D2), jnp.float32),
          jax.ShapeDtypeStruct((B,), jnp.float32),
          jax.ShapeDtypeStruct((B,), jnp.float32),
      ],
      mesh=mesh,
      scratch_types=[
          pltpu.VMEM((NNZ,), jnp.int32),
          pltpu.VMEM((NNZ,), jnp.int32),
          pltpu.VMEM((rpw,), jnp.int32),
          pltpu.VMEM((rpw,), jnp.int32),
          pltpu.VMEM((rpw,), jnp.float32),
          pltpu.VMEM((rpw, D2), jnp.float32),
          pltpu.SemaphoreType.DMA,
      ] + [pltpu.VMEM((_CHUNK,), jnp.int32) for _ in range(rpw // _CHUNK)],
      compiler_params=pltpu.CompilerParams(
          needs_layout_passes=False, use_tc_tiling_on_sc=False),
      interpret=interpret)


def _make_sc_fixed(B, D2, vhf, interpret=False):
  n_workers = _NC * _NS
  rpw = B // n_workers
  mesh = plsc.VectorSubcoreMesh(
      core_axis_name="c", subcore_axis_name="s",
      num_cores=_NC, num_subcores=_NS)
  n_chunks = rpw // _CHUNK

  def sc_body(ff, ftab, f_out, hf_out, fidx, hbuf, rows, sem, *gidx):
    wid = lax.axis_index("s") * _NC + lax.axis_index("c")
    base = wid * rpw

    with jax.named_scope("fixed_gather"):
      pltpu.sync_copy(ff.at[pl.ds(base, rpw)], fidx)
      for i in range(rpw // _LANES):
        r = fidx[pl.ds(i * _LANES, _LANES)]
        hi = r >= vhf
        lo = (i * _LANES) % _CHUNK
        gidx[(i * _LANES) // _CHUNK][pl.ds(lo, _LANES)] = jnp.where(
            hi, r - vhf, r)
        hbuf[pl.ds(i * _LANES, _LANES)] = hi.astype(jnp.float32)
      descs = [
          pltpu.async_copy(ftab.at[gidx[k]],
                           rows.at[pl.ds(k * _CHUNK, _CHUNK)], sem)
          for k in range(n_chunks)
      ]
      for d in descs:
        d.wait()
      pltpu.sync_copy(rows, f_out.at[pl.ds(base, rpw)])
      pltpu.sync_copy(hbuf, hf_out.at[pl.ds(base, rpw)])

  return pl.kernel(
      sc_body,
      out_type=[
          jax.ShapeDtypeStruct((B, D2), jnp.float32),
          jax.ShapeDtypeStruct((B,), jnp.float32),
      ],
      mesh=mesh,
      scratch_types=[
          pltpu.VMEM((rpw,), jnp.int32),
          pltpu.VMEM((rpw,), jnp.float32),
          pltpu.VMEM((rpw, D2), jnp.float32),
          pltpu.SemaphoreType.DMA,
      ] + [pltpu.VMEM((_CHUNK,), jnp.int32) for _ in range(rpw // _CHUNK)],
      compiler_params=pltpu.CompilerParams(
          needs_layout_passes=False, use_tc_tiling_on_sc=False),
      interpret=interpret)


def _make_tc_kernel(B, D, OD, blk, interpret=False):
  def tc_body(f2, g2, a2, hf, mg, ma, w, b, o):
    wv = w[:]
    hfv = hf[:]
    f = jnp.where(hfv > 0.5, f2[:, D:], f2[:, :D])
    mgv, mav = mg[:], ma[:]
    zero = jnp.zeros_like(mgv)
    g = (jnp.where(mgv == 1.0, g2[:, :D], zero)
         + jnp.where(mgv == 2.0, g2[:, D:], zero))
    a = (jnp.where(mav == 1.0, a2[:, :D], zero)
         + jnp.where(mav == 2.0, a2[:, D:], zero))
    s = g + a
    acc = jnp.dot(f, wv[:D, :], preferred_element_type=jnp.float32)
    acc = acc + jnp.dot(s, wv[D:, :], preferred_element_type=jnp.float32)
    o[:] = acc + b[:]

  return pl.pallas_call(
      tc_body,
      grid=(B // blk,),
      in_specs=[
          pl.BlockSpec((blk, 2 * D), lambda i: (i, 0)),
          pl.BlockSpec((blk, 2 * D), lambda i: (i, 0)),
          pl.BlockSpec((blk, 2 * D), lambda i: (i, 0)),
          pl.BlockSpec((blk, 1), lambda i: (i, 0)),
          pl.BlockSpec((blk, 1), lambda i: (i, 0)),
          pl.BlockSpec((blk, 1), lambda i: (i, 0)),
          pl.BlockSpec((2 * D, OD), lambda i: (0, 0)),
          pl.BlockSpec((1, OD), lambda i: (0, 0)),
      ],
      out_specs=pl.BlockSpec((blk, OD), lambda i: (i, 0)),
      out_shape=jax.ShapeDtypeStruct((B, OD), jnp.float32),
      interpret=interpret)


def _run(fixed_features, geom_index, geom_value, attr_index, attr_value,
         fixed_table, geom_table, attr_table, W, b, interpret=False):
  B = fixed_features.shape[0]
  NNZ = geom_index.shape[0]
  FV, D = fixed_table.shape
  SV = geom_table.shape[0]
  OD = W.shape[1]
  ff = fixed_features.astype(jnp.int32)
  gi = geom_index.astype(jnp.int32)
  gv = geom_value.astype(jnp.int32)
  ai = attr_index.astype(jnp.int32)
  av = attr_value.astype(jnp.int32)

  fold_f = _make_fold_kernel(FV, D, interpret)
  fold_s = _make_fold_kernel(SV, D, interpret)
  gtab2 = fold_s(geom_table.T, geom_table.T)
  atab2 = fold_s(attr_table.T, attr_table.T)
  vhf = _fold_half(FV)
  vhs = _fold_half(SV)

  g2, a2, mg, ma = _make_sc_sparse(B, NNZ, 2 * D, vhs, interpret)(
      gi, gv, ai, av, gtab2, atab2)
  ftab2 = fold_f(fixed_table.T, fixed_table.T)
  f2, hf = _make_sc_fixed(B, 2 * D, vhf, interpret)(ff, ftab2)
  return _make_tc_kernel(B, D, OD, min(2048, B), interpret)(
      f2, g2, a2, hf.reshape(B, 1), mg.reshape(B, 1), ma.reshape(B, 1),
      W, b.reshape(1, OD))


def kernel(fixed_features, geom_index, geom_value, attr_index, attr_value,
           fixed_table, geom_table, attr_table, W, b):
  return _run(fixed_features, geom_index, geom_value, attr_index, attr_value,
              fixed_table, geom_table, attr_table, W, b)


# speedup vs baseline: 3.1204x; 1.0001x over previous
import jax
import jax.numpy as jnp
from jax import lax
from jax.experimental import pallas as pl
from jax.experimental.pallas import tpu as pltpu
from jax.experimental.pallas import tpu_sc as plsc

_LANES = 16
_NC = 2
_NS = 16
_CHUNK = 128
_NB = 16384


def _fold_half(v):
  return ((v + _NB - 1) // _NB + 1) // 2 * _NB


def _make_fold_kernel(V, D, interpret=False):
  vh = _fold_half(V)
  n_lo = vh // _NB
  last = (V - 1) // _NB

  def body(x_lo, x_hi, o):
    eye = jnp.eye(D, dtype=jnp.float32)
    dn = (((0,), (0,)), ((), ()))
    o[:, :D] = lax.dot_general(x_lo[:], eye, dn,
                               preferred_element_type=jnp.float32)
    o[:, D:] = lax.dot_general(x_hi[:], eye, dn,
                               preferred_element_type=jnp.float32)

  return pl.pallas_call(
      body,
      grid=(n_lo,),
      in_specs=[pl.BlockSpec((D, _NB), lambda i: (0, i)),
                pl.BlockSpec((D, _NB),
                             lambda i: (0, jnp.minimum(n_lo + i, last)))],
      out_specs=pl.BlockSpec((_NB, 2 * D), lambda i: (i, 0)),
      out_shape=jax.ShapeDtypeStruct((vh, 2 * D), jnp.float32),
      interpret=interpret)


def _make_sc_sparse(B, NNZ, D2, vhs, interpret=False):
  n_workers = _NC * _NS
  rpw = B // n_workers
  assert B % n_workers == 0 and rpw % _CHUNK == 0 and NNZ % _LANES == 0

  mesh = plsc.VectorSubcoreMesh(
      core_axis_name="c", subcore_axis_name="s",
      num_cores=_NC, num_subcores=_NS)

  n_chunks = rpw // _CHUNK

  def sc_body(gi, gv, ai, av, gtab, atab,
              g_out, a_out, mg_out, ma_out,
              sidx, sval, valbuf, srcbuf, mbuf, rows, sem, *gidx):
    wid = lax.axis_index("s") * _NC + lax.axis_index("c")
    base = wid * rpw

    def gather_rows(tab):
      descs = [
          pltpu.async_copy(tab.at[gidx[k]],
                           rows.at[pl.ds(k * _CHUNK, _CHUNK)], sem)
          for k in range(n_chunks)
      ]
      for d in descs:
        d.wait()

    def zbody(i, _):
      srcbuf[pl.ds(i * _LANES, _LANES)] = jnp.zeros((_LANES,), jnp.int32)
      return 0
    lax.fori_loop(0, rpw // _LANES, zbody, 0)

    with jax.named_scope("winner_resolve"):
      for ih, vh_, code in ((gi, gv, 1), (ai, av, 2)):
        pltpu.sync_copy(ih, sidx)
        pltpu.sync_copy(vh_, sval)
        code16 = jnp.full((_LANES,), code, jnp.int32)

        def p1body(i, _, code16=code16):
          idx16 = sidx[pl.ds(i * _LANES, _LANES)]
          val16 = sval[pl.ds(i * _LANES, _LANES)]
          inb = (idx16 >= base) & (idx16 < base + rpw)
          local = jnp.where(inb, idx16 - base, 0)
          _, win = plsc.scan_count(local, mask=inb)
          plsc.store_scatter(valbuf, [local], val16, mask=win)
          plsc.store_scatter(srcbuf, [local], code16, mask=win)
          return 0
        lax.fori_loop(0, NNZ // _LANES, p1body, 0)

    for tab, out_hbm, m_out, code in ((gtab, g_out, mg_out, 1),
                                      (atab, a_out, ma_out, 2)):
      code16 = jnp.full((_LANES,), code, jnp.int32)

      with jax.named_scope(f"sparse_gather_{code}"):
        lane16 = lax.iota(jnp.int32, _LANES)
        for i in range(rpw // _LANES):
          v = valbuf[pl.ds(i * _LANES, _LANES)]
          s = srcbuf[pl.ds(i * _LANES, _LANES)]
          sel = s == code16
          vg = jnp.where(v >= vhs, v - vhs, v)
          lo = (i * _LANES) % _CHUNK
          gidx[(i * _LANES) // _CHUNK][pl.ds(lo, _LANES)] = jnp.where(
              sel, vg, base + lane16 + i * _LANES)
          mbuf[pl.ds(i * _LANES, _LANES)] = jnp.where(
              sel & (v < vhs), 1.0, 0.0) + jnp.where(
              sel & (v >= vhs), 2.0, 0.0)
        gather_rows(tab)
        pltpu.sync_copy(rows, out_hbm.at[pl.ds(base, rpw)])
        pltpu.sync_copy(mbuf, m_out.at[pl.ds(base, rpw)])

  return pl.kernel(
      sc_body,
      out_type=[
          jax.ShapeDtypeStruct((B, D2), jnp.float32),
          jax.ShapeDtypeStruct((B, D2), jnp.float32),
          jax.ShapeDtypeStruct((B,), jnp.float32),
          jax.ShapeDtypeStruct((B,), jnp.float32),
      ],
      mesh=mesh,
      scratch_types=[
          pltpu.VMEM((NNZ,), jnp.int32),
          pltpu.VMEM((NNZ,), jnp.int32),
          pltpu.VMEM((rpw,), jnp.int32),
          pltpu.VMEM((rpw,), jnp.int32),
          pltpu.VMEM((rpw,), jnp.float32),
          pltpu.VMEM((rpw, D2), jnp.float32),
          pltpu.SemaphoreType.DMA,
      ] + [pltpu.VMEM((_CHUNK,), jnp.int32) for _ in range(rpw // _CHUNK)],
      compiler_params=pltpu.CompilerParams(
          needs_layout_passes=False, use_tc_tiling_on_sc=False),
      interpret=interpret)


def _make_sc_fixed(B, D2, vhf, interpret=False):
  n_workers = _NC * _NS
  rpw = B // n_workers
  mesh = plsc.VectorSubcoreMesh(
      core_axis_name="c", subcore_axis_name="s",
      num_cores=_NC, num_subcores=_NS)
  n_chunks = rpw // _CHUNK

  def sc_body(ff, ftab, f_out, hf_out, fidx, hbuf, rows, sem, *gidx):
    wid = lax.axis_index("s") * _NC + lax.axis_index("c")
    base = wid * rpw

    with jax.named_scope("fixed_gather"):
      pltpu.sync_copy(ff.at[pl.ds(base, rpw)], fidx)
      for i in range(rpw // _LANES):
        r = fidx[pl.ds(i * _LANES, _LANES)]
        hi = r >= vhf
        lo = (i * _LANES) % _CHUNK
        gidx[(i * _LANES) // _CHUNK][pl.ds(lo, _LANES)] = jnp.where(
            hi, r - vhf, r)
        hbuf[pl.ds(i * _LANES, _LANES)] = hi.astype(jnp.float32)
      descs = [
          pltpu.async_copy(ftab.at[gidx[k]],
                           rows.at[pl.ds(k * _CHUNK, _CHUNK)], sem)
          for k in range(n_chunks)
      ]
      for d in descs:
        d.wait()
      pltpu.sync_copy(rows, f_out.at[pl.ds(base, rpw)])
      pltpu.sync_copy(hbuf, hf_out.at[pl.ds(base, rpw)])

  return pl.kernel(
      sc_body,
      out_type=[
          jax.ShapeDtypeStruct((B, D2), jnp.float32),
          jax.ShapeDtypeStruct((B,), jnp.float32),
      ],
      mesh=mesh,
      scratch_types=[
          pltpu.VMEM((rpw,), jnp.int32),
          pltpu.VMEM((rpw,), jnp.float32),
          pltpu.VMEM((rpw, D2), jnp.float32),
          pltpu.SemaphoreType.DMA,
      ] + [pltpu.VMEM((_CHUNK,), jnp.int32) for _ in range(rpw // _CHUNK)],
      compiler_params=pltpu.CompilerParams(
          needs_layout_passes=False, use_tc_tiling_on_sc=False),
      interpret=interpret)


def _make_tc_kernel(B, D, OD, blk, interpret=False):
  def tc_body(f2, g2, a2, hf, mg, ma, w, b, o):
    wv = w[:]
    hfv = hf[:]
    f = jnp.where(hfv > 0.5, f2[:, D:], f2[:, :D])
    mgv, mav = mg[:], ma[:]
    zero = jnp.zeros_like(mgv)
    g = (jnp.where(mgv == 1.0, g2[:, :D], zero)
         + jnp.where(mgv == 2.0, g2[:, D:], zero))
    a = (jnp.where(mav == 1.0, a2[:, :D], zero)
         + jnp.where(mav == 2.0, a2[:, D:], zero))
    s = g + a
    acc = jnp.dot(f, wv[:D, :], preferred_element_type=jnp.float32)
    acc = acc + jnp.dot(s, wv[D:, :], preferred_element_type=jnp.float32)
    o[:] = acc + b[:]

  return pl.pallas_call(
      tc_body,
      grid=(B // blk,),
      in_specs=[
          pl.BlockSpec((blk, 2 * D), lambda i: (i, 0)),
          pl.BlockSpec((blk, 2 * D), lambda i: (i, 0)),
          pl.BlockSpec((blk, 2 * D), lambda i: (i, 0)),
          pl.BlockSpec((blk, 1), lambda i: (i, 0)),
          pl.BlockSpec((blk, 1), lambda i: (i, 0)),
          pl.BlockSpec((blk, 1), lambda i: (i, 0)),
          pl.BlockSpec((2 * D, OD), lambda i: (0, 0)),
          pl.BlockSpec((1, OD), lambda i: (0, 0)),
      ],
      out_specs=pl.BlockSpec((blk, OD), lambda i: (i, 0)),
      out_shape=jax.ShapeDtypeStruct((B, OD), jnp.float32),
      interpret=interpret)


def _run(fixed_features, geom_index, geom_value, attr_index, attr_value,
         fixed_table, geom_table, attr_table, W, b, interpret=False):
  B = fixed_features.shape[0]
  NNZ = geom_index.shape[0]
  FV, D = fixed_table.shape
  SV = geom_table.shape[0]
  OD = W.shape[1]
  ff = fixed_features.astype(jnp.int32)
  gi = geom_index.astype(jnp.int32)
  gv = geom_value.astype(jnp.int32)
  ai = attr_index.astype(jnp.int32)
  av = attr_value.astype(jnp.int32)

  fold_f = _make_fold_kernel(FV, D, interpret)
  fold_s = _make_fold_kernel(SV, D, interpret)
  gtab2 = fold_s(geom_table.T, geom_table.T)
  atab2 = fold_s(attr_table.T, attr_table.T)
  vhf = _fold_half(FV)
  vhs = _fold_half(SV)

  g2, a2, mg, ma = _make_sc_sparse(B, NNZ, 2 * D, vhs, interpret)(
      gi, gv, ai, av, gtab2, atab2)
  ftab2 = fold_f(fixed_table.T, fixed_table.T)
  f2, hf = _make_sc_fixed(B, 2 * D, vhf, interpret)(ff, ftab2)
  return _make_tc_kernel(B, D, OD, min(4096, B), interpret)(
      f2, g2, a2, hf.reshape(B, 1), mg.reshape(B, 1), ma.reshape(B, 1),
      W, b.reshape(1, OD))


def kernel(fixed_features, geom_index, geom_value, attr_index, attr_value,
           fixed_table, geom_table, attr_table, W, b):
  return _run(fixed_features, geom_index, geom_value, attr_index, attr_value,
              fixed_table, geom_table, attr_table, W, b)
